# Initial kernel scaffold; baseline (speedup 1.0000x reference)
import jax
import jax.numpy as jnp
from jax.experimental import pallas as pl


def kernel(x, edge_index, W_mlp, b_mlp, W_lin, b_lin):
    raise NotImplementedError("write your pallas kernel here")


# speedup vs baseline: 1.0189x; 1.0189x over previous
import jax
import jax.numpy as jnp
from jax.experimental import pallas as pl

_P = 0.5


def _mlp_body(x_ref, w_ref, b_ref, o_ref):
    o_ref[...] = jnp.tanh(
        jnp.dot(x_ref[...], w_ref[...], preferred_element_type=jnp.float32)
        + b_ref[...]
    )


def _out_body(x_ref, hr_ref, w_ref, b_ref, o_ref):
    o_ref[...] = (
        jnp.dot(x_ref[...] + hr_ref[...], w_ref[...],
                preferred_element_type=jnp.float32)
        + b_ref[...]
    )


def kernel(x, edge_index, W_mlp, b_mlp, W_lin, b_lin):
    n, d_in = x.shape
    c = W_mlp.shape[0]
    d_out = W_lin.shape[0]
    e = edge_index.shape[1]
    src = edge_index[0]
    dst = edge_index[1]

    bm = 1000
    grid = (n // bm,)

    t = pl.pallas_call(
        _mlp_body,
        grid=grid,
        in_specs=[
            pl.BlockSpec((bm, d_in), lambda i: (i, 0)),
            pl.BlockSpec((d_in, c), lambda i: (0, 0)),
            pl.BlockSpec((1, c), lambda i: (0, 0)),
        ],
        out_specs=pl.BlockSpec((bm, c), lambda i: (i, 0)),
        out_shape=jax.ShapeDtypeStruct((n, c), jnp.float32),
    )(x, W_mlp.T, b_mlp[None])

    d = jnp.sum(jnp.abs(t[src] - t[dst]), axis=1)

    deg = jnp.bincount(dst, length=n)
    num_keep = jnp.ceil(_P * deg.astype(jnp.float32)).astype(jnp.int32)
    order = jnp.lexsort((d, dst))
    dst_s = dst[order]
    src_s = src[order]
    start = jnp.cumsum(deg) - deg
    rank = jnp.arange(e, dtype=jnp.int32) - start[dst_s].astype(jnp.int32)
    keep = rank < num_keep[dst_s]

    msg = jnp.where(keep[:, None], x[src_s], jnp.zeros((), dtype=x.dtype))
    summed = jax.ops.segment_sum(msg, dst_s, num_segments=n)
    denom = jnp.maximum(num_keep, 1).astype(x.dtype)[:, None]
    mean_h = summed / denom
    hr = jnp.where((deg > 0)[:, None], mean_h, x)

    out = pl.pallas_call(
        _out_body,
        grid=grid,
        in_specs=[
            pl.BlockSpec((bm, d_in), lambda i: (i, 0)),
            pl.BlockSpec((bm, d_in), lambda i: (i, 0)),
            pl.BlockSpec((d_in, d_out), lambda i: (0, 0)),
            pl.BlockSpec((1, d_out), lambda i: (0, 0)),
        ],
        out_specs=pl.BlockSpec((bm, d_out), lambda i: (i, 0)),
        out_shape=jax.ShapeDtypeStruct((n, d_out), jnp.float32),
    )(x, hr, W_lin.T, b_lin[None])

    return out
